# Initial kernel scaffold; baseline (speedup 1.0000x reference)
import jax
import jax.numpy as jnp
from jax.experimental import pallas as pl


def kernel(h, x, mess_graph, Wz_w, Wz_b, Wr_w, Ur_w, Ur_b, Wh_w, Wh_b):
    raise NotImplementedError("write your pallas kernel here")


# speedup vs baseline: 1.8557x; 1.8557x over previous
import functools

import jax
import jax.numpy as jnp
from jax import lax
from jax.experimental import pallas as pl
from jax.experimental.pallas import tpu as pltpu
from jax.experimental.pallas import tpu_sc as plsc

DEPTH = 3
H = 128
N = 160000
K = 8

NC = 2
NS = 16
NW = NC * NS
EPW = N // NW
C = 8
RPC = C * K
NCHUNK = EPW // C
LG = H // 16


def _sc_body(h_hbm, hu_hbm, nr1_hbm, idx_hbm, sh_hbm, sg_hbm,
             idx0, idx1, h0, h1, u0, u1, r0, r1, sh_v, sg_v, sem0, sem1):
    w = lax.axis_index("s") * NC + lax.axis_index("c")
    ebase = w * EPW

    bufs = ((idx0, h0, u0, r0, sem0), (idx1, h1, u1, r1, sem1))

    def start(j, b):
        idxb, hb, ub, rb, semb = bufs[b]
        e0 = ebase + j * C
        pltpu.sync_copy(idx_hbm.at[pl.ds(e0 * K, RPC)], idxb)
        pltpu.async_copy(h_hbm.at[idxb], hb, semb)
        pltpu.async_copy(hu_hbm.at[idxb], ub, semb)
        pltpu.async_copy(nr1_hbm.at[pl.ds(e0, C)], rb, semb)

    def wait(b):
        idxb, hb, ub, rb, semb = bufs[b]
        pltpu.make_async_copy(h_hbm.at[idxb], hb, semb).wait()
        pltpu.make_async_copy(hu_hbm.at[idxb], ub, semb).wait()
        pltpu.make_async_copy(nr1_hbm.at[pl.ds(0, C)], rb, semb).wait()

    def compute(j, b):
        _, hb, ub, rb, _ = bufs[b]

        def edge_body(e, carry):
            for g in range(LG):
                sl = pl.ds(g * 16, 16)
                nr = rb[e, sl]
                acc_h = jnp.zeros((16,), jnp.float32)
                acc_g = jnp.zeros((16,), jnp.float32)
                for k in range(K):
                    row = e * K + k
                    hv = hb[row, sl]
                    uv = ub[row, sl]
                    ex = jnp.exp(nr - uv)
                    s = 1.0 / (1.0 + ex)
                    acc_h = acc_h + hv
                    acc_g = acc_g + s * hv
                sh_v[e, sl] = acc_h
                sg_v[e, sl] = acc_g
            return carry

        lax.fori_loop(0, C, edge_body, 0)
        e0 = ebase + j * C
        pltpu.sync_copy(sh_v, sh_hbm.at[pl.ds(e0, C)])
        pltpu.sync_copy(sg_v, sg_hbm.at[pl.ds(e0, C)])

    start(0, 0)
    start(1, 1)

    def chunk_pair(jj, carry):
        for b in range(2):
            j = jj * 2 + b
            wait(b)
            compute(j, b)
            nxt = j + 2

            @pl.when(nxt < NCHUNK)
            def _():
                start(nxt, b)
        return carry

    lax.fori_loop(0, NCHUNK // 2, chunk_pair, 0)
    if NCHUNK % 2:
        wait(0)
        compute(NCHUNK - 1, 0)


_sc_gather = functools.partial(
    pl.kernel,
    mesh=plsc.VectorSubcoreMesh(core_axis_name="c", subcore_axis_name="s"),
    out_type=[
        jax.ShapeDtypeStruct((N, H), jnp.float32),
        jax.ShapeDtypeStruct((N, H), jnp.float32),
    ],
    scratch_types=[
        pltpu.VMEM((RPC,), jnp.int32),
        pltpu.VMEM((RPC,), jnp.int32),
        pltpu.VMEM((RPC, H), jnp.float32),
        pltpu.VMEM((RPC, H), jnp.float32),
        pltpu.VMEM((RPC, H), jnp.float32),
        pltpu.VMEM((RPC, H), jnp.float32),
        pltpu.VMEM((C, H), jnp.float32),
        pltpu.VMEM((C, H), jnp.float32),
        pltpu.VMEM((C, H), jnp.float32),
        pltpu.VMEM((C, H), jnp.float32),
        pltpu.SemaphoreType.DMA,
        pltpu.SemaphoreType.DMA,
    ],
)(_sc_body)


BLK = 2000
GRID = N // BLK


def _pre_body(x_ref, h_ref, wzx, whx, wr, ur, wzb, whb, urb,
              xz_o, xh_o, nr1_o, hu_o):
    xv = x_ref[...]
    hv = h_ref[...]
    f32 = jnp.float32
    xz_o[...] = jnp.dot(xv, wzx[...], preferred_element_type=f32) + wzb[...]
    xh_o[...] = jnp.dot(xv, whx[...], preferred_element_type=f32) + whb[...]
    nr1_o[...] = -jnp.dot(xv, wr[...], preferred_element_type=f32) - urb[...]
    hu_o[...] = jnp.dot(hv, ur[...], preferred_element_type=f32)


_pre = pl.pallas_call(
    _pre_body,
    grid=(GRID,),
    in_specs=[
        pl.BlockSpec((BLK, H), lambda i: (i, 0)),
        pl.BlockSpec((BLK, H), lambda i: (i, 0)),
        pl.BlockSpec((H, H), lambda i: (0, 0)),
        pl.BlockSpec((H, H), lambda i: (0, 0)),
        pl.BlockSpec((H, H), lambda i: (0, 0)),
        pl.BlockSpec((H, H), lambda i: (0, 0)),
        pl.BlockSpec((1, H), lambda i: (0, 0)),
        pl.BlockSpec((1, H), lambda i: (0, 0)),
        pl.BlockSpec((1, H), lambda i: (0, 0)),
    ],
    out_specs=[pl.BlockSpec((BLK, H), lambda i: (i, 0))] * 4,
    out_shape=[jax.ShapeDtypeStruct((N, H), jnp.float32)] * 4,
)


def _comb_body(sh_ref, sg_ref, xz_ref, xh_ref, wzh, whh, ur, h_o, hu_o):
    f32 = jnp.float32
    sh = sh_ref[...]
    sg = sg_ref[...]
    z = jax.nn.sigmoid(xz_ref[...] + jnp.dot(sh, wzh[...], preferred_element_type=f32))
    p = jnp.tanh(xh_ref[...] + jnp.dot(sg, whh[...], preferred_element_type=f32))
    hn = (1.0 - z) * sh + z * p
    rows = lax.broadcasted_iota(jnp.int32, (BLK, 1), 0) + pl.program_id(0) * BLK
    hn = jnp.where(rows == 0, 0.0, hn)
    h_o[...] = hn
    hu_o[...] = jnp.dot(hn, ur[...], preferred_element_type=f32)


_comb = pl.pallas_call(
    _comb_body,
    grid=(GRID,),
    in_specs=[
        pl.BlockSpec((BLK, H), lambda i: (i, 0)),
        pl.BlockSpec((BLK, H), lambda i: (i, 0)),
        pl.BlockSpec((BLK, H), lambda i: (i, 0)),
        pl.BlockSpec((BLK, H), lambda i: (i, 0)),
        pl.BlockSpec((H, H), lambda i: (0, 0)),
        pl.BlockSpec((H, H), lambda i: (0, 0)),
        pl.BlockSpec((H, H), lambda i: (0, 0)),
    ],
    out_specs=[pl.BlockSpec((BLK, H), lambda i: (i, 0))] * 2,
    out_shape=[jax.ShapeDtypeStruct((N, H), jnp.float32)] * 2,
)


def kernel(h, x, mess_graph, Wz_w, Wz_b, Wr_w, Ur_w, Ur_b, Wh_w, Wh_b):
    idx = mess_graph.astype(jnp.int32).reshape(-1)
    WzxT = Wz_w[:, :H].T
    WzhT = Wz_w[:, H:].T
    WhxT = Wh_w[:, :H].T
    WhhT = Wh_w[:, H:].T
    WrT = Wr_w.T
    UrT = Ur_w.T
    wzb = Wz_b.reshape(1, H)
    whb = Wh_b.reshape(1, H)
    urb = Ur_b.reshape(1, H)

    Xz, Xh, nr1, hU = _pre(x, h, WzxT, WhxT, WrT, UrT, wzb, whb, urb)
    for _d in range(DEPTH):
        sum_h, sum_g = _sc_gather(h, hU, nr1, idx)
        h, hU = _comb(sum_h, sum_g, Xz, Xh, WzhT, WhhT, UrT)
    return h
